# baseline (speedup 1.0000x reference)
import functools

import jax
import jax.numpy as jnp
from jax import lax
from jax.experimental import pallas as pl
from jax.experimental.pallas import tpu as pltpu
from jax.experimental.pallas import tpu_sc as plsc

_B, _N, _J = 16, 20000, 17
_S = 4800
_DP = _J * 3
_L = 16
_NC, _NS = 2, 16
_NW = _NC * _NS
_BLK = 80
_NBLK = _S // _BLK
_ROUNDS = -(-_NBLK // _NW)
_G = _BLK // _L


def _make_gather():
    mesh = plsc.VectorSubcoreMesh(core_axis_name="c", subcore_axis_name="s")

    @functools.partial(
        pl.kernel,
        mesh=mesh,
        compiler_params=pltpu.CompilerParams(use_tc_tiling_on_sc=False),
        out_type=(
            jax.ShapeDtypeStruct((_S,), jnp.float32),
            jax.ShapeDtypeStruct((_NBLK, 4, _BLK), jnp.float32),
            jax.ShapeDtypeStruct((_S,), jnp.float32),
            jax.ShapeDtypeStruct((_NBLK, _DP, _BLK), jnp.float32),
        ),
        scratch_types=[
            pltpu.VMEM((_BLK,), jnp.int32),
            pltpu.VMEM((_BLK,), jnp.int32),
            pltpu.VMEM((_BLK,), jnp.int32),
            pltpu.VMEM((_ROUNDS, _BLK), jnp.int32),
            pltpu.VMEM((_ROUNDS, 4, _BLK), jnp.int32),
            pltpu.VMEM((_ROUNDS, _DP, _BLK), jnp.int32),
            pltpu.VMEM((_ROUNDS, _BLK), jnp.float32),
            pltpu.VMEM((_ROUNDS, 4, _BLK), jnp.float32),
            pltpu.VMEM((_ROUNDS, _BLK), jnp.float32),
            pltpu.VMEM((_ROUNDS, _DP, _BLK), jnp.float32),
            pltpu.SemaphoreType.DMA,
        ],
    )
    def gather_kernel(boxes_hbm, scores_hbm, joints_hbm,
                      selb_hbm, sell_hbm, selx_hbm,
                      out_b, out_boxes, out_sc, out_pose,
                      b_v, l_v, x_v, sidx_v, bidx_v, pidx_v,
                      bf_v, boxg_v, scg_v, poseg_v, sem):
        wid = lax.axis_index("s") * _NC + lax.axis_index("c")
        blks = [jnp.minimum(wid + _NW * t, _NBLK - 1) for t in range(_ROUNDS)]
        pend = [[] for _ in range(_ROUNDS)]
        for t in range(_ROUNDS):
            base = blks[t] * _BLK
            pltpu.sync_copy(selb_hbm.at[pl.ds(base, _BLK)], b_v)
            pltpu.sync_copy(sell_hbm.at[pl.ds(base, _BLK)], l_v)
            pltpu.sync_copy(selx_hbm.at[pl.ds(base, _BLK)], x_v)
            for i in range(_G):
                sl = pl.ds(i * _L, _L)
                b = b_v[sl]
                x = x_v[sl]
                flat = b * _N + x
                bf_v[t, sl] = b.astype(jnp.float32)
                sidx_v[t, sl] = flat + l_v[sl] * _N
                fb = b * (4 * _N) + x
                for c in range(4):
                    bidx_v[t, c, sl] = fb + c * _N
                for c in range(_DP):
                    pidx_v[t, c, sl] = flat + c * (_B * _N)
            pend[t].append(pltpu.async_copy(
                scores_hbm.at[sidx_v.at[t]], scg_v.at[t], sem))
            for c in range(4):
                pend[t].append(pltpu.async_copy(
                    boxes_hbm.at[bidx_v.at[t, c]], boxg_v.at[t, c], sem))
            for c in range(_DP):
                pend[t].append(pltpu.async_copy(
                    joints_hbm.at[pidx_v.at[t, c]], poseg_v.at[t, c], sem))

        for t in range(_ROUNDS):
            base = blks[t] * _BLK
            for d in pend[t]:
                d.wait()
            pltpu.sync_copy(bf_v.at[t], out_b.at[pl.ds(base, _BLK)])
            pltpu.sync_copy(boxg_v.at[t], out_boxes.at[blks[t]])
            pltpu.sync_copy(scg_v.at[t], out_sc.at[pl.ds(base, _BLK)])
            pltpu.sync_copy(poseg_v.at[t], out_pose.at[blks[t]])

    return gather_kernel


_gather = _make_gather()


@jax.jit
def kernel(pred_boxes, pred_scores, pred_joints, selected_indexes):
    boxes_flat = jnp.transpose(pred_boxes, (0, 2, 1)).reshape(_B * 4 * _N)
    scores_flat = jnp.transpose(pred_scores, (0, 2, 1)).reshape(_B * _N)
    joints_flat = jnp.transpose(pred_joints, (2, 3, 0, 1)).reshape(
        _DP * _B * _N)
    bf, boxes_t, sc, pose_t = _gather(
        boxes_flat, scores_flat, joints_flat,
        selected_indexes[:, 0], selected_indexes[:, 1], selected_indexes[:, 2])
    boxes = boxes_t.transpose(0, 2, 1).reshape(_S, 4)
    pose = pose_t.transpose(0, 2, 1).reshape(_S, _DP)
    return jnp.concatenate([bf[:, None], boxes, sc[:, None], pose], axis=1)


# speedup vs baseline: 1.0029x; 1.0029x over previous
import functools

import jax
import jax.numpy as jnp
from jax import lax
from jax.experimental import pallas as pl
from jax.experimental.pallas import tpu as pltpu
from jax.experimental.pallas import tpu_sc as plsc

_B, _N, _J = 16, 20000, 17
_S = 4800
_DP = _J * 3
_L = 16
_NC, _NS = 2, 16
_NW = _NC * _NS
_BLK = 80
_NBLK = _S // _BLK
_ROUNDS = -(-_NBLK // _NW)
_G = _BLK // _L


def _make_gather():
    mesh = plsc.VectorSubcoreMesh(core_axis_name="c", subcore_axis_name="s")

    @functools.partial(
        pl.kernel,
        mesh=mesh,
        compiler_params=pltpu.CompilerParams(use_tc_tiling_on_sc=False),
        out_type=(
            jax.ShapeDtypeStruct((_S,), jnp.float32),
            jax.ShapeDtypeStruct((_NBLK, 4 * _BLK), jnp.float32),
            jax.ShapeDtypeStruct((_S,), jnp.float32),
            jax.ShapeDtypeStruct((_NBLK, _DP * _BLK), jnp.float32),
        ),
        scratch_types=[
            pltpu.VMEM((_BLK,), jnp.int32),
            pltpu.VMEM((_BLK,), jnp.int32),
            pltpu.VMEM((_BLK,), jnp.int32),
            pltpu.VMEM((_ROUNDS, _BLK), jnp.int32),
            pltpu.VMEM((_ROUNDS, 4 * _BLK), jnp.int32),
            pltpu.VMEM((_ROUNDS, _DP * _BLK), jnp.int32),
            pltpu.VMEM((_ROUNDS, _BLK), jnp.float32),
            pltpu.VMEM((_ROUNDS, 4 * _BLK), jnp.float32),
            pltpu.VMEM((_ROUNDS, _BLK), jnp.float32),
            pltpu.VMEM((_ROUNDS, _DP * _BLK), jnp.float32),
            pltpu.SemaphoreType.DMA,
        ],
    )
    def gather_kernel(boxes_hbm, scores_hbm, joints_hbm,
                      selb_hbm, sell_hbm, selx_hbm,
                      out_b, out_boxes, out_sc, out_pose,
                      b_v, l_v, x_v, sidx_v, bidx_v, pidx_v,
                      bf_v, boxg_v, scg_v, poseg_v, sem):
        wid = lax.axis_index("s") * _NC + lax.axis_index("c")
        blks = [jnp.minimum(wid + _NW * t, _NBLK - 1) for t in range(_ROUNDS)]
        pend = [[] for _ in range(_ROUNDS)]
        for t in range(_ROUNDS):
            base = blks[t] * _BLK
            pltpu.sync_copy(selb_hbm.at[pl.ds(base, _BLK)], b_v)
            pltpu.sync_copy(sell_hbm.at[pl.ds(base, _BLK)], l_v)
            pltpu.sync_copy(selx_hbm.at[pl.ds(base, _BLK)], x_v)
            for i in range(_G):
                sl = pl.ds(i * _L, _L)
                b = b_v[sl]
                x = x_v[sl]
                flat = b * _N + x
                bf_v[t, sl] = b.astype(jnp.float32)
                sidx_v[t, sl] = flat + l_v[sl] * _N
                fb = b * (4 * _N) + x
                for c in range(4):
                    bidx_v[t, pl.ds(c * _BLK + i * _L, _L)] = fb + c * _N
                for c in range(_DP):
                    pidx_v[t, pl.ds(c * _BLK + i * _L, _L)] = (
                        flat + c * (_B * _N))
            pend[t] = [
                pltpu.async_copy(scores_hbm.at[sidx_v.at[t]],
                                 scg_v.at[t], sem),
                pltpu.async_copy(boxes_hbm.at[bidx_v.at[t]],
                                 boxg_v.at[t], sem),
                pltpu.async_copy(joints_hbm.at[pidx_v.at[t]],
                                 poseg_v.at[t], sem),
            ]

        for t in range(_ROUNDS):
            base = blks[t] * _BLK
            for d in pend[t]:
                d.wait()
            pltpu.sync_copy(bf_v.at[t], out_b.at[pl.ds(base, _BLK)])
            pltpu.sync_copy(boxg_v.at[t], out_boxes.at[blks[t]])
            pltpu.sync_copy(scg_v.at[t], out_sc.at[pl.ds(base, _BLK)])
            pltpu.sync_copy(poseg_v.at[t], out_pose.at[blks[t]])

    return gather_kernel


_gather = _make_gather()


@jax.jit
def kernel(pred_boxes, pred_scores, pred_joints, selected_indexes):
    boxes_flat = jnp.transpose(pred_boxes, (0, 2, 1)).reshape(_B * 4 * _N)
    scores_flat = jnp.transpose(pred_scores, (0, 2, 1)).reshape(_B * _N)
    joints_flat = jnp.transpose(pred_joints, (2, 3, 0, 1)).reshape(
        _DP * _B * _N)
    bf, boxes_t, sc, pose_t = _gather(
        boxes_flat, scores_flat, joints_flat,
        selected_indexes[:, 0], selected_indexes[:, 1], selected_indexes[:, 2])
    boxes = boxes_t.reshape(_NBLK, 4, _BLK).transpose(0, 2, 1).reshape(_S, 4)
    pose = pose_t.reshape(_NBLK, _DP, _BLK).transpose(0, 2, 1).reshape(_S, _DP)
    return jnp.concatenate([bf[:, None], boxes, sc[:, None], pose], axis=1)


# speedup vs baseline: 1.0219x; 1.0189x over previous
import functools

import jax
import jax.numpy as jnp
from jax import lax
from jax.experimental import pallas as pl
from jax.experimental.pallas import tpu as pltpu
from jax.experimental.pallas import tpu_sc as plsc

_B, _N, _J = 16, 20000, 17
_S = 4800
_DP = _J * 3
_W = 2 + 4 + _DP
_L = 16
_NC, _NS = 2, 16
_NW = _NC * _NS
_BLK = 80
_NBLK = _S // _BLK
_ROUNDS = -(-_NBLK // _NW)
_G = _BLK // _L


def _make_gather():
    mesh = plsc.VectorSubcoreMesh(core_axis_name="c", subcore_axis_name="s")

    @functools.partial(
        pl.kernel,
        mesh=mesh,
        compiler_params=pltpu.CompilerParams(use_tc_tiling_on_sc=False),
        out_type=jax.ShapeDtypeStruct((_NBLK, _W * _BLK), jnp.float32),
        scratch_types=[
            pltpu.VMEM((_BLK,), jnp.int32),
            pltpu.VMEM((_BLK,), jnp.int32),
            pltpu.VMEM((_BLK,), jnp.int32),
            pltpu.VMEM((_ROUNDS, _BLK), jnp.int32),
            pltpu.VMEM((_ROUNDS, 4 * _BLK), jnp.int32),
            pltpu.VMEM((_ROUNDS, _DP * _BLK), jnp.int32),
            pltpu.VMEM((_ROUNDS, _W * _BLK), jnp.float32),
            pltpu.SemaphoreType.DMA,
        ],
    )
    def gather_kernel(boxes_hbm, scores_hbm, joints_hbm,
                      selb_hbm, sell_hbm, selx_hbm, out_all,
                      b_v, l_v, x_v, sidx_v, bidx_v, pidx_v, allg_v, sem):
        wid = lax.axis_index("s") * _NC + lax.axis_index("c")
        blks = [jnp.minimum(wid + _NW * t, _NBLK - 1) for t in range(_ROUNDS)]
        pend = [[] for _ in range(_ROUNDS)]
        for t in range(_ROUNDS):
            base = blks[t] * _BLK
            pltpu.sync_copy(selb_hbm.at[pl.ds(base, _BLK)], b_v)
            pltpu.sync_copy(sell_hbm.at[pl.ds(base, _BLK)], l_v)
            pltpu.sync_copy(selx_hbm.at[pl.ds(base, _BLK)], x_v)
            for i in range(_G):
                sl = pl.ds(i * _L, _L)
                b = b_v[sl]
                x = x_v[sl]
                flat = b * _N + x
                allg_v[t, pl.ds(i * _L, _L)] = b.astype(jnp.float32)
                sidx_v[t, sl] = flat + l_v[sl] * _N
                fb = b * (4 * _N) + x
                for c in range(4):
                    bidx_v[t, pl.ds(c * _BLK + i * _L, _L)] = fb + c * _N
                for c in range(_DP):
                    pidx_v[t, pl.ds(c * _BLK + i * _L, _L)] = (
                        flat + c * (_B * _N))
            pend[t] = [
                pltpu.async_copy(scores_hbm.at[sidx_v.at[t]],
                                 allg_v.at[t, pl.ds(5 * _BLK, _BLK)], sem),
                pltpu.async_copy(boxes_hbm.at[bidx_v.at[t]],
                                 allg_v.at[t, pl.ds(_BLK, 4 * _BLK)], sem),
                pltpu.async_copy(joints_hbm.at[pidx_v.at[t]],
                                 allg_v.at[t, pl.ds(6 * _BLK, _DP * _BLK)],
                                 sem),
            ]

        for t in range(_ROUNDS):
            for d in pend[t]:
                d.wait()
            pltpu.sync_copy(allg_v.at[t], out_all.at[blks[t]])

    return gather_kernel


_gather = _make_gather()


@jax.jit
def kernel(pred_boxes, pred_scores, pred_joints, selected_indexes):
    boxes_flat = jnp.transpose(pred_boxes, (0, 2, 1)).reshape(_B * 4 * _N)
    scores_flat = jnp.transpose(pred_scores, (0, 2, 1)).reshape(_B * _N)
    joints_flat = jnp.transpose(pred_joints, (2, 3, 0, 1)).reshape(
        _DP * _B * _N)
    out_all = _gather(
        boxes_flat, scores_flat, joints_flat,
        selected_indexes[:, 0], selected_indexes[:, 1], selected_indexes[:, 2])
    return (out_all.reshape(_NBLK, _W, _BLK)
            .transpose(0, 2, 1).reshape(_S, _W))
